# baseline (speedup 1.0000x reference)
import functools

import jax
import jax.numpy as jnp
from jax import lax
from jax.experimental import pallas as pl
from jax.experimental.pallas import tpu as pltpu
from jax.experimental.pallas import tpu_sc as plsc

_LAMBDA = 0.1
_B = 1024
_C = 100000

_NC, _NS, _L = 2, 16, 16
_NW = _NC * _NS
_BPW = _B // _NW


@functools.cache
def _make_gather_kernel():
    mesh = plsc.VectorSubcoreMesh(core_axis_name="c", subcore_axis_name="s")

    @functools.partial(
        pl.kernel,
        mesh=mesh,
        out_type=jax.ShapeDtypeStruct((_NW, _L), jnp.float32),
        scratch_types=[
            pltpu.VMEM((_BPW,), jnp.int32),
            pltpu.VMEM((_BPW,), jnp.int32),
            pltpu.VMEM((_BPW,), jnp.float32),
            pltpu.VMEM((_L,), jnp.float32),
            pltpu.SemaphoreType.DMA,
        ],
    )
    def gather_kernel(flat_hbm, tgt_hbm, out_hbm, tgt_v, idx_v, vals_v, part_v, sem):
        wid = lax.axis_index("s") * _NC + lax.axis_index("c")
        base = wid * _BPW
        pltpu.sync_copy(tgt_hbm.at[pl.ds(base, _BPW)], tgt_v)
        for j in range(_BPW // _L):
            rows = (base + j * _L) + lax.iota(jnp.int32, _L)
            idx_v[pl.ds(j * _L, _L)] = rows * jnp.int32(_C) + tgt_v[pl.ds(j * _L, _L)]
        pltpu.async_copy(flat_hbm.at[idx_v], vals_v, sem).wait()
        part = vals_v[pl.ds(0, _L)]
        for j in range(1, _BPW // _L):
            part = part + vals_v[pl.ds(j * _L, _L)]
        part_v[...] = part
        pltpu.sync_copy(part_v, out_hbm.at[wid])

    return gather_kernel


_SUM_ROWS = 3200
_SUM_COLS = 32000
_SUM_BR = 32
_SUM_GRID = _SUM_ROWS // _SUM_BR


def _sum_body(x_ref, o_ref):
    @pl.when(pl.program_id(0) == 0)
    def _init():
        o_ref[0, 0] = 0.0

    o_ref[0, 0] += jnp.sum(x_ref[...])


def kernel(output, target):
    flat = output.reshape(-1)
    partials = _make_gather_kernel()(flat, target.astype(jnp.int32))

    flat2d = output.reshape(_SUM_ROWS, _SUM_COLS)
    tsum = pl.pallas_call(
        _sum_body,
        grid=(_SUM_GRID,),
        in_specs=[pl.BlockSpec((_SUM_BR, _SUM_COLS), lambda i: (i, 0))],
        out_specs=pl.BlockSpec(memory_space=pltpu.SMEM),
        out_shape=jax.ShapeDtypeStruct((1, 1), jnp.float32),
    )(flat2d)[0, 0]

    jsum = jnp.sum(partials)
    return (jsum - tsum / _C) * (_LAMBDA / _B)


# speedup vs baseline: 3.0712x; 3.0712x over previous
import jax
import jax.numpy as jnp
from jax.experimental import pallas as pl
from jax.experimental.pallas import tpu as pltpu

_LAMBDA = 0.1
_B = 1024
_C = 100000

_BR = 8
_GRID = _B // _BR


def _body(x_ref, t_ref, tsum_ref, jsum_ref):
    @pl.when(pl.program_id(0) == 0)
    def _init():
        tsum_ref[0, 0] = 0.0
        jsum_ref[0, 0] = 0.0

    x = x_ref[...]
    t = t_ref[...]
    col = jax.lax.broadcasted_iota(jnp.int32, (_BR, _C), 1)
    tsum_ref[0, 0] += jnp.sum(x)
    jsum_ref[0, 0] += jnp.sum(jnp.where(col == t, x, 0.0))


def kernel(output, target):
    tgt2d = target.astype(jnp.int32).reshape(_B, 1)
    tsum, jsum = pl.pallas_call(
        _body,
        grid=(_GRID,),
        in_specs=[
            pl.BlockSpec((_BR, _C), lambda i: (i, 0)),
            pl.BlockSpec((_BR, 1), lambda i: (i, 0)),
        ],
        out_specs=[
            pl.BlockSpec(memory_space=pltpu.SMEM),
            pl.BlockSpec(memory_space=pltpu.SMEM),
        ],
        out_shape=[
            jax.ShapeDtypeStruct((1, 1), jnp.float32),
            jax.ShapeDtypeStruct((1, 1), jnp.float32),
        ],
    )(output, tgt2d)
    return (jsum[0, 0] - tsum[0, 0] / _C) * (_LAMBDA / _B)


# speedup vs baseline: 9.3278x; 3.0372x over previous
import jax
import jax.numpy as jnp
from jax.experimental import pallas as pl
from jax.experimental.pallas import tpu as pltpu

_LAMBDA = 0.1
_B = 1024
_C = 100000

_BC = 1000
_GRID = _C // _BC


def _body(x_ref, t_ref, tsum_ref, jsum_ref):
    @pl.when(pl.program_id(0) == 0)
    def _init():
        tsum_ref[0, 0] = 0.0
        jsum_ref[0, 0] = 0.0

    x = x_ref[...]
    t = t_ref[0]
    cls = pl.program_id(0) * _BC + jax.lax.broadcasted_iota(jnp.int32, (_BC, _B), 0)
    tsum_ref[0, 0] += jnp.sum(x)
    jsum_ref[0, 0] += jnp.sum(jnp.where(cls == t, x, 0.0))


def kernel(output, target):
    xt = output.T
    tgt3d = target.astype(jnp.int32).reshape(1, 1, _B)
    tsum, jsum = pl.pallas_call(
        _body,
        grid=(_GRID,),
        in_specs=[
            pl.BlockSpec((_BC, _B), lambda i: (i, 0)),
            pl.BlockSpec((1, 1, _B), lambda i: (0, 0, 0)),
        ],
        out_specs=[
            pl.BlockSpec(memory_space=pltpu.SMEM),
            pl.BlockSpec(memory_space=pltpu.SMEM),
        ],
        out_shape=[
            jax.ShapeDtypeStruct((1, 1), jnp.float32),
            jax.ShapeDtypeStruct((1, 1), jnp.float32),
        ],
    )(xt, tgt3d)
    return (jsum[0, 0] - tsum[0, 0] / _C) * (_LAMBDA / _B)


# speedup vs baseline: 13.2445x; 1.4199x over previous
import jax
import jax.numpy as jnp
from jax.experimental import pallas as pl

_LAMBDA = 0.1
_B = 1024
_C = 100000

_BC = 2000
_GRID = _C // _BC


def _body(x_ref, t_ref, tacc_ref, jacc_ref):
    @pl.when(pl.program_id(0) == 0)
    def _init():
        tacc_ref[...] = jnp.zeros((8, _B), jnp.float32)
        jacc_ref[...] = jnp.zeros((8, _B), jnp.float32)

    x = x_ref[...]
    t = t_ref[0]
    cls = pl.program_id(0) * _BC + jax.lax.broadcasted_iota(jnp.int32, (_BC, _B), 0)
    sel = jnp.where(cls == t, x, 0.0)
    tacc_ref[...] += jnp.sum(x.reshape(_BC // 8, 8, _B), axis=0)
    jacc_ref[...] += jnp.sum(sel.reshape(_BC // 8, 8, _B), axis=0)


def kernel(output, target):
    xt = output.T
    tgt3d = target.astype(jnp.int32).reshape(1, 1, _B)
    tacc, jacc = pl.pallas_call(
        _body,
        grid=(_GRID,),
        in_specs=[
            pl.BlockSpec((_BC, _B), lambda i: (i, 0)),
            pl.BlockSpec((1, 1, _B), lambda i: (0, 0, 0)),
        ],
        out_specs=[
            pl.BlockSpec((8, _B), lambda i: (0, 0)),
            pl.BlockSpec((8, _B), lambda i: (0, 0)),
        ],
        out_shape=[
            jax.ShapeDtypeStruct((8, _B), jnp.float32),
            jax.ShapeDtypeStruct((8, _B), jnp.float32),
        ],
    )(xt, tgt3d)
    return (jnp.sum(jacc) - jnp.sum(tacc) / _C) * (_LAMBDA / _B)


# speedup vs baseline: 13.6424x; 1.0300x over previous
import jax
import jax.numpy as jnp
from jax.experimental import pallas as pl

_LAMBDA = 0.1
_B = 1024
_C = 100000

_BC = 4000
_GRID = _C // _BC


def _body(x_ref, t_ref, tacc_ref, jacc_ref):
    @pl.when(pl.program_id(0) == 0)
    def _init():
        tacc_ref[...] = jnp.zeros((8, _B), jnp.float32)
        jacc_ref[...] = jnp.zeros((8, _B), jnp.float32)

    x = x_ref[...]
    t = t_ref[0]
    cls = pl.program_id(0) * _BC + jax.lax.broadcasted_iota(jnp.int32, (_BC, _B), 0)
    sel = jnp.where(cls == t, x, 0.0)
    tacc_ref[...] += jnp.sum(x.reshape(_BC // 8, 8, _B), axis=0)
    jacc_ref[...] += jnp.sum(sel.reshape(_BC // 8, 8, _B), axis=0)


def kernel(output, target):
    xt = output.T
    tgt3d = target.astype(jnp.int32).reshape(1, 1, _B)
    tacc, jacc = pl.pallas_call(
        _body,
        grid=(_GRID,),
        in_specs=[
            pl.BlockSpec((_BC, _B), lambda i: (i, 0)),
            pl.BlockSpec((1, 1, _B), lambda i: (0, 0, 0)),
        ],
        out_specs=[
            pl.BlockSpec((8, _B), lambda i: (0, 0)),
            pl.BlockSpec((8, _B), lambda i: (0, 0)),
        ],
        out_shape=[
            jax.ShapeDtypeStruct((8, _B), jnp.float32),
            jax.ShapeDtypeStruct((8, _B), jnp.float32),
        ],
    )(xt, tgt3d)
    return (jnp.sum(jacc) - jnp.sum(tacc) / _C) * (_LAMBDA / _B)
